# Initial kernel scaffold; baseline (speedup 1.0000x reference)
import jax
import jax.numpy as jnp
from jax.experimental import pallas as pl


def kernel(node_attr, edge_attr, edge_idx, Wn, b_in, Wf, bf, gh, bh, go, bo, We, b_out):
    raise NotImplementedError("write your pallas kernel here")


# speedup vs baseline: 28.6875x; 28.6875x over previous
import functools

import jax
import jax.numpy as jnp
import numpy as np
from jax import lax
from jax.experimental import pallas as pl
from jax.experimental.pallas import tpu as pltpu
from jax.experimental.pallas import tpu_sc as plsc

_EPS = 1e-5
_NPB = 800
_NPB2 = 2000


def _sc_gather(table, idx):
    V, D = table.shape
    E = idx.shape[0]
    info = plsc.get_sparse_core_info()
    NC, NS = info.num_cores, info.num_subcores
    NW = NC * NS
    assert E % NW == 0
    e_per_w = E // NW
    CH = 2000
    assert e_per_w % CH == 0
    n_ch = e_per_w // CH
    mesh = plsc.VectorSubcoreMesh(core_axis_name="c", subcore_axis_name="s")

    @functools.partial(
        pl.kernel,
        mesh=mesh,
        out_type=jax.ShapeDtypeStruct((E, D), jnp.float32),
        compiler_params=pltpu.CompilerParams(use_tc_tiling_on_sc=False),
        scratch_types=[
            pltpu.VMEM((CH,), jnp.int32),
            pltpu.VMEM((CH, D), jnp.float32),
            pltpu.SemaphoreType.DMA,
        ],
    )
    def k(table_hbm, idx_hbm, out_hbm, idx_v, rows_v, sem):
        wid = lax.axis_index("s") * NC + lax.axis_index("c")
        base = wid * e_per_w

        def body(i, carry):
            off = base + i * CH
            pltpu.sync_copy(idx_hbm.at[pl.ds(off, CH)], idx_v)
            pltpu.async_copy(table_hbm.at[idx_v], rows_v, sem).wait()
            pltpu.sync_copy(rows_v, out_hbm.at[pl.ds(off, CH)])
            return carry

        lax.fori_loop(0, n_ch, body, 0)

    return k(table, idx)


def _tc_init(na, wnrow, binrow, wsF, wsC, bfF, bfC):
    BN = na.shape[0]
    HA = wnrow.shape[1]
    grid = BN // _NPB2

    def body(na_ref, w_ref, b_ref, wsF_ref, wsC_ref, bfF_ref, bfC_ref,
             atom_ref, spF_ref, spC_ref):
        a = na_ref[...] * w_ref[...] + b_ref[...]
        atom_ref[...] = a
        spF_ref[...] = (
            jnp.dot(a, wsF_ref[...], preferred_element_type=jnp.float32) + bfF_ref[...]
        )
        spC_ref[...] = (
            jnp.dot(a, wsC_ref[...], preferred_element_type=jnp.float32) + bfC_ref[...]
        )

    return pl.pallas_call(
        body,
        grid=(grid,),
        in_specs=[
            pl.BlockSpec((_NPB2, 1), lambda i: (i, 0)),
            pl.BlockSpec((1, HA), lambda i: (0, 0)),
            pl.BlockSpec((1, HA), lambda i: (0, 0)),
            pl.BlockSpec((HA, 128), lambda i: (0, 0)),
            pl.BlockSpec((HA, 128), lambda i: (0, 0)),
            pl.BlockSpec((1, 128), lambda i: (0, 0)),
            pl.BlockSpec((1, 128), lambda i: (0, 0)),
        ],
        out_specs=(
            pl.BlockSpec((_NPB2, HA), lambda i: (i, 0)),
            pl.BlockSpec((_NPB2, 128), lambda i: (i, 0)),
            pl.BlockSpec((_NPB2, 128), lambda i: (i, 0)),
        ),
        out_shape=(
            jax.ShapeDtypeStruct((BN, HA), jnp.float32),
            jax.ShapeDtypeStruct((BN, 128), jnp.float32),
            jax.ShapeDtypeStruct((BN, 128), jnp.float32),
        ),
    )(na, wnrow, binrow, wsF, wsC, bfF, bfC)


def _tc_pass_a(anbr3, ef3, spF, spC, wnFk, wnCk, weFk, weCk):
    BN = spF.shape[0]
    grid = BN // _NPB

    def body(a_ref, e_ref, spF_ref, spC_ref,
             wnF_ref, wnC_ref, weF_ref, weC_ref, st_ref):
        i = pl.program_id(0)
        sF = spF_ref[...]
        sC = spC_ref[...]
        s1F = s2F = s1C = s2C = None
        for k in range(4):
            a2 = a_ref[k]
            e2 = e_ref[k]
            gF = (
                jnp.dot(a2, wnF_ref[...], preferred_element_type=jnp.float32)
                + jnp.dot(e2, weF_ref[...], preferred_element_type=jnp.float32)
                + sF
            )
            gC = (
                jnp.dot(a2, wnC_ref[...], preferred_element_type=jnp.float32)
                + jnp.dot(e2, weC_ref[...], preferred_element_type=jnp.float32)
                + sC
            )
            p1F = jnp.sum(gF, axis=0, keepdims=True)
            p2F = jnp.sum(gF * gF, axis=0, keepdims=True)
            p1C = jnp.sum(gC, axis=0, keepdims=True)
            p2C = jnp.sum(gC * gC, axis=0, keepdims=True)
            if s1F is None:
                s1F, s2F, s1C, s2C = p1F, p2F, p1C, p2C
            else:
                s1F, s2F, s1C, s2C = s1F + p1F, s2F + p2F, s1C + p1C, s2C + p2C
        upd = jnp.concatenate([s1F, s2F, s1C, s2C], axis=0)

        @pl.when(i == 0)
        def _():
            st_ref[...] = jnp.zeros_like(st_ref)

        st_ref[...] += upd

    return pl.pallas_call(
        body,
        grid=(grid,),
        in_specs=[
            pl.BlockSpec((4, _NPB, 128), lambda i: (0, i, 0)),
            pl.BlockSpec((4, _NPB, 128), lambda i: (0, i, 0)),
            pl.BlockSpec((_NPB, 128), lambda i: (i, 0)),
            pl.BlockSpec((_NPB, 128), lambda i: (i, 0)),
            pl.BlockSpec((128, 128), lambda i: (0, 0)),
            pl.BlockSpec((128, 128), lambda i: (0, 0)),
            pl.BlockSpec((128, 128), lambda i: (0, 0)),
            pl.BlockSpec((128, 128), lambda i: (0, 0)),
        ],
        out_specs=pl.BlockSpec((4, 128), lambda i: (0, 0)),
        out_shape=jax.ShapeDtypeStruct((4, 128), jnp.float32),
        compiler_params=pltpu.CompilerParams(dimension_semantics=("arbitrary",)),
    )(anbr3, ef3, spF, spC, wnFk, wnCk, weFk, weCk)


def _tc_pass_b(anbr3, ef3, spF, spC, wnFk, wnCk, weFk, weCk, st,
               ghFt, bhFt, ghCt, bhCt, Kfold, Tfold, S):
    BN = spF.shape[0]
    HA = Tfold.shape[1]
    grid = BN // _NPB

    def body(a_ref, e_ref, spF_ref, spC_ref,
             wnF_ref, wnC_ref, weF_ref, weC_ref, st_ref,
             ghF_ref, bhF_ref, ghC_ref, bhC_ref, K_ref, T_ref,
             sm_ref, st2_ref):
        i = pl.program_id(0)
        stf = jnp.dot(st_ref[...], K_ref[...], preferred_element_type=jnp.float32)
        m1F = stf[0:1, :] / S
        vF = stf[1:2, :] / S - m1F * m1F
        aF = ghF_ref[...] * lax.rsqrt(vF + _EPS)
        cF = bhF_ref[...] - m1F * aF
        m1C = stf[2:3, :] / S
        vC = stf[3:4, :] / S - m1C * m1C
        aC = ghC_ref[...] * lax.rsqrt(vC + _EPS)
        cC = bhC_ref[...] - m1C * aC
        sF = spF_ref[...]
        sC = spC_ref[...]
        tot = None
        for k in range(4):
            a2 = a_ref[k]
            e2 = e_ref[k]
            gF = (
                jnp.dot(a2, wnF_ref[...], preferred_element_type=jnp.float32)
                + jnp.dot(e2, weF_ref[...], preferred_element_type=jnp.float32)
                + sF
            ) * aF + cF
            gC = (
                jnp.dot(a2, wnC_ref[...], preferred_element_type=jnp.float32)
                + jnp.dot(e2, weC_ref[...], preferred_element_type=jnp.float32)
                + sC
            ) * aC + cC
            p = jax.nn.sigmoid(gF) * jnp.maximum(gC, 0.0)
            tot = p if tot is None else tot + p
        sm = jnp.dot(tot, T_ref[...], preferred_element_type=jnp.float32)
        sm_ref[...] = sm
        t1 = jnp.sum(sm, axis=0, keepdims=True)
        t2 = jnp.sum(sm * sm, axis=0, keepdims=True)

        @pl.when(i == 0)
        def _():
            st2_ref[...] = jnp.zeros_like(st2_ref)

        st2_ref[...] += jnp.concatenate([t1, t2], axis=0)

    return pl.pallas_call(
        body,
        grid=(grid,),
        in_specs=[
            pl.BlockSpec((4, _NPB, 128), lambda i: (0, i, 0)),
            pl.BlockSpec((4, _NPB, 128), lambda i: (0, i, 0)),
            pl.BlockSpec((_NPB, 128), lambda i: (i, 0)),
            pl.BlockSpec((_NPB, 128), lambda i: (i, 0)),
            pl.BlockSpec((128, 128), lambda i: (0, 0)),
            pl.BlockSpec((128, 128), lambda i: (0, 0)),
            pl.BlockSpec((128, 128), lambda i: (0, 0)),
            pl.BlockSpec((128, 128), lambda i: (0, 0)),
            pl.BlockSpec((4, 128), lambda i: (0, 0)),
            pl.BlockSpec((1, 128), lambda i: (0, 0)),
            pl.BlockSpec((1, 128), lambda i: (0, 0)),
            pl.BlockSpec((1, 128), lambda i: (0, 0)),
            pl.BlockSpec((1, 128), lambda i: (0, 0)),
            pl.BlockSpec((128, 128), lambda i: (0, 0)),
            pl.BlockSpec((128, HA), lambda i: (0, 0)),
        ],
        out_specs=(
            pl.BlockSpec((_NPB, HA), lambda i: (i, 0)),
            pl.BlockSpec((2, HA), lambda i: (0, 0)),
        ),
        out_shape=(
            jax.ShapeDtypeStruct((BN, HA), jnp.float32),
            jax.ShapeDtypeStruct((2, HA), jnp.float32),
        ),
        compiler_params=pltpu.CompilerParams(dimension_semantics=("arbitrary",)),
    )(anbr3, ef3, spF, spC, wnFk, wnCk, weFk, weCk, st,
      ghFt, bhFt, ghCt, bhCt, Kfold, Tfold)


def _tc_bn2sp(atom, sm, st2, go2, bo2, wsF, wsC, bfF, bfC):
    BN, HA = atom.shape
    Sn = float(BN)
    grid = BN // _NPB2

    def body(a_ref, sm_ref, st2_ref, go_ref, bo_ref,
             wsF_ref, wsC_ref, bfF_ref, bfC_ref,
             atom_ref, spF_ref, spC_ref):
        st_v = st2_ref[...]
        m1 = st_v[0:1, :] / Sn
        v = st_v[1:2, :] / Sn - m1 * m1
        aa = go_ref[...] * lax.rsqrt(v + _EPS)
        cc = bo_ref[...] - m1 * aa
        na_ = jnp.maximum(a_ref[...] + aa * sm_ref[...] + cc, 0.0)
        atom_ref[...] = na_
        spF_ref[...] = (
            jnp.dot(na_, wsF_ref[...], preferred_element_type=jnp.float32) + bfF_ref[...]
        )
        spC_ref[...] = (
            jnp.dot(na_, wsC_ref[...], preferred_element_type=jnp.float32) + bfC_ref[...]
        )

    return pl.pallas_call(
        body,
        grid=(grid,),
        in_specs=[
            pl.BlockSpec((_NPB2, HA), lambda i: (i, 0)),
            pl.BlockSpec((_NPB2, HA), lambda i: (i, 0)),
            pl.BlockSpec((2, HA), lambda i: (0, 0)),
            pl.BlockSpec((1, HA), lambda i: (0, 0)),
            pl.BlockSpec((1, HA), lambda i: (0, 0)),
            pl.BlockSpec((HA, 128), lambda i: (0, 0)),
            pl.BlockSpec((HA, 128), lambda i: (0, 0)),
            pl.BlockSpec((1, 128), lambda i: (0, 0)),
            pl.BlockSpec((1, 128), lambda i: (0, 0)),
        ],
        out_specs=(
            pl.BlockSpec((_NPB2, HA), lambda i: (i, 0)),
            pl.BlockSpec((_NPB2, 128), lambda i: (i, 0)),
            pl.BlockSpec((_NPB2, 128), lambda i: (i, 0)),
        ),
        out_shape=(
            jax.ShapeDtypeStruct((BN, HA), jnp.float32),
            jax.ShapeDtypeStruct((BN, 128), jnp.float32),
            jax.ShapeDtypeStruct((BN, 128), jnp.float32),
        ),
    )(atom, sm, st2, go2, bo2, wsF, wsC, bfF, bfC)


def _tc_bn2_final(atom, sm, st2, go2, bo2, werow, b0):
    BN, HA = atom.shape
    Sn = float(BN)
    grid = BN // _NPB2

    def body(a_ref, sm_ref, st2_ref, go_ref, bo_ref, we_ref, b0_ref, out_ref):
        i = pl.program_id(0)
        st_v = st2_ref[...]
        m1 = st_v[0:1, :] / Sn
        v = st_v[1:2, :] / Sn - m1 * m1
        aa = go_ref[...] * lax.rsqrt(v + _EPS)
        cc = bo_ref[...] - m1 * aa
        na_ = jnp.maximum(a_ref[...] + aa * sm_ref[...] + cc, 0.0)
        val = jnp.sum(na_ * we_ref[...])

        @pl.when(i == 0)
        def _():
            out_ref[...] = Sn * b0_ref[...]

        out_ref[...] += val.reshape(1, 1)

    return pl.pallas_call(
        body,
        grid=(grid,),
        in_specs=[
            pl.BlockSpec((_NPB2, HA), lambda i: (i, 0)),
            pl.BlockSpec((_NPB2, HA), lambda i: (i, 0)),
            pl.BlockSpec((2, HA), lambda i: (0, 0)),
            pl.BlockSpec((1, HA), lambda i: (0, 0)),
            pl.BlockSpec((1, HA), lambda i: (0, 0)),
            pl.BlockSpec((1, HA), lambda i: (0, 0)),
            pl.BlockSpec((1, 1), lambda i: (0, 0)),
        ],
        out_specs=pl.BlockSpec((1, 1), lambda i: (0, 0)),
        out_shape=jax.ShapeDtypeStruct((1, 1), jnp.float32),
        compiler_params=pltpu.CompilerParams(dimension_semantics=("arbitrary",)),
    )(atom, sm, st2, go2, bo2, werow, b0)


def kernel(node_attr, edge_attr, edge_idx, Wn, b_in, Wf, bf, gh, bh, go, bo, We, b_out):
    B, N, M = edge_idx.shape
    HA = Wn.shape[0]
    HB = edge_attr.shape[-1]
    BN = B * N
    E = BN * M
    MG = M // 4

    f32 = jnp.float32
    eye4 = jnp.eye(4, dtype=f32)

    na = node_attr.reshape(BN, 1)

    idx_off = edge_idx + (jnp.arange(B, dtype=edge_idx.dtype) * N)[:, None, None]
    idx_r = idx_off.reshape(BN, MG, 4).transpose(1, 0, 2).reshape(E)

    ef4 = edge_attr.reshape(BN, MG, 4, HB).transpose(1, 0, 2, 3)
    ef4 = jnp.concatenate(
        [ef4, jnp.zeros((MG, BN, 4, HA - HB), dtype=f32)], axis=-1
    )
    ef3 = ef4.reshape(MG, BN, 128)

    r128 = np.arange(128)
    Kfold = jnp.asarray((r128[:, None] % HA == r128[None, :] % HA), dtype=f32)
    Tfold = jnp.asarray((r128[:, None] % HA == np.arange(HA)[None, :]), dtype=f32)

    def tile4(x):
        return jnp.concatenate([x] * 4, axis=1)

    n_layers = Wf.shape[0]
    layers = []
    for i in range(n_layers):
        Wfi = Wf[i]
        wsF = jnp.concatenate([Wfi[:HA, :HA].T] * 4, axis=1)
        wsC = jnp.concatenate([Wfi[HA:, :HA].T] * 4, axis=1)
        wnFk = jnp.kron(eye4, Wfi[:HA, HA : 2 * HA].T)
        wnCk = jnp.kron(eye4, Wfi[HA:, HA : 2 * HA].T)
        weF_pad = jnp.concatenate(
            [Wfi[:HA, 2 * HA :].T, jnp.zeros((HA - HB, HA), dtype=f32)], axis=0
        )
        weC_pad = jnp.concatenate(
            [Wfi[HA:, 2 * HA :].T, jnp.zeros((HA - HB, HA), dtype=f32)], axis=0
        )
        weFk = jnp.kron(eye4, weF_pad)
        weCk = jnp.kron(eye4, weC_pad)
        layers.append(dict(
            wsF=wsF, wsC=wsC, wnFk=wnFk, wnCk=wnCk, weFk=weFk, weCk=weCk,
            bfF=tile4(bf[i][:HA].reshape(1, HA)),
            bfC=tile4(bf[i][HA:].reshape(1, HA)),
            ghFt=tile4(gh[i][:HA].reshape(1, HA)),
            ghCt=tile4(gh[i][HA:].reshape(1, HA)),
            bhFt=tile4(bh[i][:HA].reshape(1, HA)),
            bhCt=tile4(bh[i][HA:].reshape(1, HA)),
            go2=go[i].reshape(1, HA),
            bo2=bo[i].reshape(1, HA),
        ))

    L0 = layers[0]
    atom, spF, spC = _tc_init(
        na, Wn.reshape(1, HA), b_in.reshape(1, HA),
        L0["wsF"], L0["wsC"], L0["bfF"], L0["bfC"],
    )

    out = None
    for i in range(n_layers):
        Li = layers[i]
        anbr3 = _sc_gather(atom, idx_r).reshape(MG, BN, 128)
        st = _tc_pass_a(
            anbr3, ef3, spF, spC, Li["wnFk"], Li["wnCk"], Li["weFk"], Li["weCk"]
        )
        sm, st2 = _tc_pass_b(
            anbr3, ef3, spF, spC, Li["wnFk"], Li["wnCk"], Li["weFk"], Li["weCk"],
            st, Li["ghFt"], Li["bhFt"], Li["ghCt"], Li["bhCt"], Kfold, Tfold,
            float(E),
        )
        if i < n_layers - 1:
            Ln = layers[i + 1]
            atom, spF, spC = _tc_bn2sp(
                atom, sm, st2, Li["go2"], Li["bo2"],
                Ln["wsF"], Ln["wsC"], Ln["bfF"], Ln["bfC"],
            )
        else:
            out = _tc_bn2_final(
                atom, sm, st2, Li["go2"], Li["bo2"],
                We.reshape(1, HA), b_out.reshape(1, 1),
            )

    return out.reshape(())


# speedup vs baseline: 28.7424x; 1.0019x over previous
import functools

import jax
import jax.numpy as jnp
import numpy as np
from jax import lax
from jax.experimental import pallas as pl
from jax.experimental.pallas import tpu as pltpu
from jax.experimental.pallas import tpu_sc as plsc

_EPS = 1e-5
_NPB = 800
_NPB2 = 2000


def _sc_gather(table, idx):
    V, D = table.shape
    E = idx.shape[0]
    info = plsc.get_sparse_core_info()
    NC, NS = info.num_cores, info.num_subcores
    NW = NC * NS
    assert E % NW == 0
    e_per_w = E // NW
    CH = 2000
    assert e_per_w % CH == 0
    n_ch = e_per_w // CH
    mesh = plsc.VectorSubcoreMesh(core_axis_name="c", subcore_axis_name="s")

    @functools.partial(
        pl.kernel,
        mesh=mesh,
        out_type=jax.ShapeDtypeStruct((E, D), jnp.float32),
        compiler_params=pltpu.CompilerParams(use_tc_tiling_on_sc=False),
        scratch_types=[
            pltpu.VMEM((CH,), jnp.int32),
            pltpu.VMEM((CH, D), jnp.float32),
            pltpu.SemaphoreType.DMA,
        ],
    )
    def k(table_hbm, idx_hbm, out_hbm, idx_v, rows_v, sem):
        wid = lax.axis_index("s") * NC + lax.axis_index("c")
        base = wid * e_per_w

        def body(i, carry):
            off = base + i * CH
            pltpu.sync_copy(idx_hbm.at[pl.ds(off, CH)], idx_v)
            pltpu.async_copy(table_hbm.at[idx_v], rows_v, sem).wait()
            pltpu.sync_copy(rows_v, out_hbm.at[pl.ds(off, CH)])
            return carry

        lax.fori_loop(0, n_ch, body, 0)

    return k(table, idx)


def _tc_init(na, wnrow, binrow, wsF, wsC, bfF, bfC):
    BN = na.shape[0]
    HA = wnrow.shape[1]
    grid = BN // _NPB2

    def body(na_ref, w_ref, b_ref, wsF_ref, wsC_ref, bfF_ref, bfC_ref,
             atom_ref, spF_ref, spC_ref):
        a = na_ref[...] * w_ref[...] + b_ref[...]
        atom_ref[...] = a
        spF_ref[...] = (
            jnp.dot(a, wsF_ref[...], preferred_element_type=jnp.float32) + bfF_ref[...]
        )
        spC_ref[...] = (
            jnp.dot(a, wsC_ref[...], preferred_element_type=jnp.float32) + bfC_ref[...]
        )

    return pl.pallas_call(
        body,
        grid=(grid,),
        in_specs=[
            pl.BlockSpec((_NPB2, 1), lambda i: (i, 0)),
            pl.BlockSpec((1, HA), lambda i: (0, 0)),
            pl.BlockSpec((1, HA), lambda i: (0, 0)),
            pl.BlockSpec((HA, 128), lambda i: (0, 0)),
            pl.BlockSpec((HA, 128), lambda i: (0, 0)),
            pl.BlockSpec((1, 128), lambda i: (0, 0)),
            pl.BlockSpec((1, 128), lambda i: (0, 0)),
        ],
        out_specs=(
            pl.BlockSpec((_NPB2, HA), lambda i: (i, 0)),
            pl.BlockSpec((_NPB2, 128), lambda i: (i, 0)),
            pl.BlockSpec((_NPB2, 128), lambda i: (i, 0)),
        ),
        out_shape=(
            jax.ShapeDtypeStruct((BN, HA), jnp.float32),
            jax.ShapeDtypeStruct((BN, 128), jnp.float32),
            jax.ShapeDtypeStruct((BN, 128), jnp.float32),
        ),
    )(na, wnrow, binrow, wsF, wsC, bfF, bfC)


def _tc_layer(anbr3, ef3, spF, spC, atom, wnFk, wnCk, weFk, weCk,
              ghFt, bhFt, ghCt, bhCt, go2, bo2, Kfold, Tfold, S,
              final, extras):
    BN, HA = atom.shape
    grid = BN // _NPB
    Sn = float(BN)

    c0 = lambda k, i: (0, 0)
    c0_3 = lambda k, i: (0, 0, 0)
    edge_map = lambda k, i: (0, jnp.where(k < 2, i, 0), 0)
    sp_map = lambda k, i: (jnp.where(k < 2, i, 0), 0)
    node_map = lambda k, i: (jnp.where(k == 2, i, 0), 0)

    def body(a_ref, e_ref, spF_ref, spC_ref, atom_ref,
             wnF_ref, wnC_ref, weF_ref, weC_ref,
             ghF_ref, bhF_ref, ghC_ref, bhC_ref,
             go_ref, bo_ref, K_ref, T_ref, *rest):
        if final:
            we_ref, b0_ref, out_ref, st_s, sm_s, st2_s = rest
        else:
            wsF_ref, wsC_ref, bfF_ref, bfC_ref, \
                natom_ref, nspF_ref, nspC_ref, st_s, sm_s, st2_s = rest
        k = pl.program_id(0)
        i = pl.program_id(1)

        def gate_halves(scale=None, shift=None):
            sF = spF_ref[...]
            sC = spC_ref[...]
            outs = []
            for kk in range(4):
                a2 = a_ref[kk]
                e2 = e_ref[kk]
                gF = (
                    jnp.dot(a2, wnF_ref[...], preferred_element_type=jnp.float32)
                    + jnp.dot(e2, weF_ref[...], preferred_element_type=jnp.float32)
                    + sF
                )
                gC = (
                    jnp.dot(a2, wnC_ref[...], preferred_element_type=jnp.float32)
                    + jnp.dot(e2, weC_ref[...], preferred_element_type=jnp.float32)
                    + sC
                )
                outs.append((gF, gC))
            return outs

        @pl.when(k == 0)
        def _phase_stats():
            s1F = s2F = s1C = s2C = None
            for gF, gC in gate_halves():
                p1F = jnp.sum(gF, axis=0, keepdims=True)
                p2F = jnp.sum(gF * gF, axis=0, keepdims=True)
                p1C = jnp.sum(gC, axis=0, keepdims=True)
                p2C = jnp.sum(gC * gC, axis=0, keepdims=True)
                if s1F is None:
                    s1F, s2F, s1C, s2C = p1F, p2F, p1C, p2C
                else:
                    s1F, s2F, s1C, s2C = s1F + p1F, s2F + p2F, s1C + p1C, s2C + p2C
            upd = jnp.concatenate([s1F, s2F, s1C, s2C], axis=0)

            @pl.when(i == 0)
            def _():
                st_s[...] = jnp.zeros_like(st_s)

            st_s[...] += upd

        @pl.when(k == 1)
        def _phase_apply():
            stf = jnp.dot(st_s[...], K_ref[...], preferred_element_type=jnp.float32)
            m1F = stf[0:1, :] / S
            vF = stf[1:2, :] / S - m1F * m1F
            aF = ghF_ref[...] * lax.rsqrt(vF + _EPS)
            cF = bhF_ref[...] - m1F * aF
            m1C = stf[2:3, :] / S
            vC = stf[3:4, :] / S - m1C * m1C
            aC = ghC_ref[...] * lax.rsqrt(vC + _EPS)
            cC = bhC_ref[...] - m1C * aC
            tot = None
            for gF, gC in gate_halves():
                p = jax.nn.sigmoid(gF * aF + cF) * jnp.maximum(gC * aC + cC, 0.0)
                tot = p if tot is None else tot + p
            sm = jnp.dot(tot, T_ref[...], preferred_element_type=jnp.float32)
            sm_s[pl.ds(i * _NPB, _NPB), :] = sm
            t1 = jnp.sum(sm, axis=0, keepdims=True)
            t2 = jnp.sum(sm * sm, axis=0, keepdims=True)

            @pl.when(i == 0)
            def _():
                st2_s[...] = jnp.zeros_like(st2_s)

            st2_s[...] += jnp.concatenate([t1, t2], axis=0)

        @pl.when(k == 2)
        def _phase_node():
            st_v = st2_s[...]
            m1 = st_v[0:1, :] / Sn
            v = st_v[1:2, :] / Sn - m1 * m1
            aa = go_ref[...] * lax.rsqrt(v + _EPS)
            cc = bo_ref[...] - m1 * aa
            sm = sm_s[pl.ds(i * _NPB, _NPB), :]
            na_ = jnp.maximum(atom_ref[...] + aa * sm + cc, 0.0)
            if final:
                val = jnp.sum(na_ * we_ref[...])

                @pl.when(i == 0)
                def _():
                    out_ref[...] = Sn * b0_ref[...]

                out_ref[...] += val.reshape(1, 1)
            else:
                natom_ref[...] = na_
                nspF_ref[...] = (
                    jnp.dot(na_, wsF_ref[...], preferred_element_type=jnp.float32)
                    + bfF_ref[...]
                )
                nspC_ref[...] = (
                    jnp.dot(na_, wsC_ref[...], preferred_element_type=jnp.float32)
                    + bfC_ref[...]
                )

    in_specs = [
        pl.BlockSpec((4, _NPB, 128), edge_map),
        pl.BlockSpec((4, _NPB, 128), edge_map),
        pl.BlockSpec((_NPB, 128), sp_map),
        pl.BlockSpec((_NPB, 128), sp_map),
        pl.BlockSpec((_NPB, HA), node_map),
        pl.BlockSpec((128, 128), c0),
        pl.BlockSpec((128, 128), c0),
        pl.BlockSpec((128, 128), c0),
        pl.BlockSpec((128, 128), c0),
        pl.BlockSpec((1, 128), c0),
        pl.BlockSpec((1, 128), c0),
        pl.BlockSpec((1, 128), c0),
        pl.BlockSpec((1, 128), c0),
        pl.BlockSpec((1, HA), c0),
        pl.BlockSpec((1, HA), c0),
        pl.BlockSpec((128, 128), c0),
        pl.BlockSpec((128, HA), c0),
    ]
    args = [anbr3, ef3, spF, spC, atom, wnFk, wnCk, weFk, weCk,
            ghFt, bhFt, ghCt, bhCt, go2, bo2, Kfold, Tfold]
    if final:
        werow, b0 = extras
        in_specs += [pl.BlockSpec((1, HA), c0), pl.BlockSpec((1, 1), c0)]
        args += [werow, b0]
        out_specs = pl.BlockSpec((1, 1), c0)
        out_shape = jax.ShapeDtypeStruct((1, 1), jnp.float32)
    else:
        wsF_n, wsC_n, bfF_n, bfC_n = extras
        in_specs += [
            pl.BlockSpec((HA, 128), c0),
            pl.BlockSpec((HA, 128), c0),
            pl.BlockSpec((1, 128), c0),
            pl.BlockSpec((1, 128), c0),
        ]
        args += [wsF_n, wsC_n, bfF_n, bfC_n]
        out_specs = (
            pl.BlockSpec((_NPB, HA), node_map),
            pl.BlockSpec((_NPB, 128), node_map),
            pl.BlockSpec((_NPB, 128), node_map),
        )
        out_shape = (
            jax.ShapeDtypeStruct((BN, HA), jnp.float32),
            jax.ShapeDtypeStruct((BN, 128), jnp.float32),
            jax.ShapeDtypeStruct((BN, 128), jnp.float32),
        )

    return pl.pallas_call(
        body,
        grid=(3, grid),
        in_specs=in_specs,
        out_specs=out_specs,
        out_shape=out_shape,
        scratch_shapes=[
            pltpu.VMEM((4, 128), jnp.float32),
            pltpu.VMEM((BN, HA), jnp.float32),
            pltpu.VMEM((2, HA), jnp.float32),
        ],
        compiler_params=pltpu.CompilerParams(
            dimension_semantics=("arbitrary", "arbitrary")
        ),
    )(*args)


def kernel(node_attr, edge_attr, edge_idx, Wn, b_in, Wf, bf, gh, bh, go, bo, We, b_out):
    B, N, M = edge_idx.shape
    HA = Wn.shape[0]
    HB = edge_attr.shape[-1]
    BN = B * N
    E = BN * M
    MG = M // 4

    f32 = jnp.float32
    eye4 = jnp.eye(4, dtype=f32)

    na = node_attr.reshape(BN, 1)

    idx_off = edge_idx + (jnp.arange(B, dtype=edge_idx.dtype) * N)[:, None, None]
    idx_r = idx_off.reshape(BN, MG, 4).transpose(1, 0, 2).reshape(E)

    ef4 = edge_attr.reshape(BN, MG, 4, HB).transpose(1, 0, 2, 3)
    ef4 = jnp.concatenate(
        [ef4, jnp.zeros((MG, BN, 4, HA - HB), dtype=f32)], axis=-1
    )
    ef3 = ef4.reshape(MG, BN, 128)

    r128 = np.arange(128)
    Kfold = jnp.asarray((r128[:, None] % HA == r128[None, :] % HA), dtype=f32)
    Tfold = jnp.asarray((r128[:, None] % HA == np.arange(HA)[None, :]), dtype=f32)

    def tile4(x):
        return jnp.concatenate([x] * 4, axis=1)

    n_layers = Wf.shape[0]
    layers = []
    for i in range(n_layers):
        Wfi = Wf[i]
        wsF = jnp.concatenate([Wfi[:HA, :HA].T] * 4, axis=1)
        wsC = jnp.concatenate([Wfi[HA:, :HA].T] * 4, axis=1)
        wnFk = jnp.kron(eye4, Wfi[:HA, HA : 2 * HA].T)
        wnCk = jnp.kron(eye4, Wfi[HA:, HA : 2 * HA].T)
        weF_pad = jnp.concatenate(
            [Wfi[:HA, 2 * HA :].T, jnp.zeros((HA - HB, HA), dtype=f32)], axis=0
        )
        weC_pad = jnp.concatenate(
            [Wfi[HA:, 2 * HA :].T, jnp.zeros((HA - HB, HA), dtype=f32)], axis=0
        )
        weFk = jnp.kron(eye4, weF_pad)
        weCk = jnp.kron(eye4, weC_pad)
        layers.append(dict(
            wsF=wsF, wsC=wsC, wnFk=wnFk, wnCk=wnCk, weFk=weFk, weCk=weCk,
            bfF=tile4(bf[i][:HA].reshape(1, HA)),
            bfC=tile4(bf[i][HA:].reshape(1, HA)),
            ghFt=tile4(gh[i][:HA].reshape(1, HA)),
            ghCt=tile4(gh[i][HA:].reshape(1, HA)),
            bhFt=tile4(bh[i][:HA].reshape(1, HA)),
            bhCt=tile4(bh[i][HA:].reshape(1, HA)),
            go2=go[i].reshape(1, HA),
            bo2=bo[i].reshape(1, HA),
        ))

    L0 = layers[0]
    atom, spF, spC = _tc_init(
        na, Wn.reshape(1, HA), b_in.reshape(1, HA),
        L0["wsF"], L0["wsC"], L0["bfF"], L0["bfC"],
    )

    out = None
    for i in range(n_layers):
        Li = layers[i]
        anbr3 = _sc_gather(atom, idx_r).reshape(MG, BN, 128)
        last = i == n_layers - 1
        if last:
            extras = (We.reshape(1, HA), b_out.reshape(1, 1))
        else:
            Ln = layers[i + 1]
            extras = (Ln["wsF"], Ln["wsC"], Ln["bfF"], Ln["bfC"])
        res = _tc_layer(
            anbr3, ef3, spF, spC, atom,
            Li["wnFk"], Li["wnCk"], Li["weFk"], Li["weCk"],
            Li["ghFt"], Li["bhFt"], Li["ghCt"], Li["bhCt"],
            Li["go2"], Li["bo2"], Kfold, Tfold, float(E),
            last, extras,
        )
        if last:
            out = res
        else:
            atom, spF, spC = res

    return out.reshape(())


# speedup vs baseline: 32.5798x; 1.1335x over previous
import functools

import jax
import jax.numpy as jnp
import numpy as np
from jax import lax
from jax.experimental import pallas as pl
from jax.experimental.pallas import tpu as pltpu
from jax.experimental.pallas import tpu_sc as plsc

_EPS = 1e-5
_NPB = 800
_NPB2 = 2000


def _sc_gather(table, idx):
    V, D = table.shape
    E = idx.shape[0]
    info = plsc.get_sparse_core_info()
    NC, NS = info.num_cores, info.num_subcores
    NW = NC * NS
    assert E % NW == 0
    e_per_w = E // NW
    CH = 2000
    assert e_per_w % CH == 0
    n_ch = e_per_w // CH
    mesh = plsc.VectorSubcoreMesh(core_axis_name="c", subcore_axis_name="s")

    @functools.partial(
        pl.kernel,
        mesh=mesh,
        out_type=jax.ShapeDtypeStruct((E, D), jnp.float32),
        compiler_params=pltpu.CompilerParams(use_tc_tiling_on_sc=False),
        scratch_types=[
            pltpu.VMEM((CH,), jnp.int32),
            pltpu.VMEM((CH, D), jnp.float32),
            pltpu.SemaphoreType.DMA,
        ],
    )
    def k(table_hbm, idx_hbm, out_hbm, idx_v, rows_v, sem):
        wid = lax.axis_index("s") * NC + lax.axis_index("c")
        base = wid * e_per_w

        def body(i, carry):
            off = base + i * CH
            pltpu.sync_copy(idx_hbm.at[pl.ds(off, CH)], idx_v)
            pltpu.async_copy(table_hbm.at[idx_v], rows_v, sem).wait()
            pltpu.sync_copy(rows_v, out_hbm.at[pl.ds(off, CH)])
            return carry

        lax.fori_loop(0, n_ch, body, 0)

    return k(table, idx)


def _tc_init(na, wnrow, binrow):
    BN = na.shape[0]
    HA = wnrow.shape[1]
    grid = BN // _NPB2

    def body(na_ref, w_ref, b_ref, atom_ref):
        atom_ref[...] = na_ref[...] * w_ref[...] + b_ref[...]

    return pl.pallas_call(
        body,
        grid=(grid,),
        in_specs=[
            pl.BlockSpec((_NPB2, 1), lambda i: (i, 0)),
            pl.BlockSpec((1, HA), lambda i: (0, 0)),
            pl.BlockSpec((1, HA), lambda i: (0, 0)),
        ],
        out_specs=pl.BlockSpec((_NPB2, HA), lambda i: (i, 0)),
        out_shape=jax.ShapeDtypeStruct((BN, HA), jnp.float32),
    )(na, wnrow, binrow)


def _tc_layer(anbr3, ef3, atom, wsF, wsC, bfF, bfC,
              wnFk, wnCk, weF0, weF1, weC0, weC1,
              ghFt, bhFt, ghCt, bhCt, go2, bo2, Kfold, Tfold, S,
              final, extras):
    BN, HA = atom.shape
    grid = BN // _NPB
    Sn = float(BN)

    c0 = lambda k, i: (0, 0)
    edge_map = lambda k, i: (0, jnp.where(k < 2, i, 0), 0)
    node_map = lambda k, i: (i, 0)
    out_map = lambda k, i: (jnp.where(k == 2, i, 0), 0)

    def body(a_ref, e_ref, atom_ref, wsF_ref, wsC_ref, bfF_ref, bfC_ref,
             wnF_ref, wnC_ref, weF0_ref, weF1_ref, weC0_ref, weC1_ref,
             ghF_ref, bhF_ref, ghC_ref, bhC_ref,
             go_ref, bo_ref, K_ref, T_ref, *rest):
        if final:
            we_ref, b0_ref, out_ref, st_s, sm_s, st2_s = rest
        else:
            natom_ref, st_s, sm_s, st2_s = rest
        k = pl.program_id(0)
        i = pl.program_id(1)

        def gate_halves():
            at = atom_ref[...]
            sF = (
                jnp.dot(at, wsF_ref[...], preferred_element_type=jnp.float32)
                + bfF_ref[...]
            )
            sC = (
                jnp.dot(at, wsC_ref[...], preferred_element_type=jnp.float32)
                + bfC_ref[...]
            )
            eh = [e_ref[0], e_ref[1]]
            weF = [weF0_ref[...], weF1_ref[...]]
            weC = [weC0_ref[...], weC1_ref[...]]
            outs = []
            for kk in range(4):
                a2 = a_ref[kk]
                e2 = eh[kk // 2]
                gF = (
                    jnp.dot(a2, wnF_ref[...], preferred_element_type=jnp.float32)
                    + jnp.dot(e2, weF[kk % 2], preferred_element_type=jnp.float32)
                    + sF
                )
                gC = (
                    jnp.dot(a2, wnC_ref[...], preferred_element_type=jnp.float32)
                    + jnp.dot(e2, weC[kk % 2], preferred_element_type=jnp.float32)
                    + sC
                )
                outs.append((gF, gC))
            return outs

        @pl.when(k == 0)
        def _phase_stats():
            s1F = s2F = s1C = s2C = None
            for gF, gC in gate_halves():
                p1F = jnp.sum(gF, axis=0, keepdims=True)
                p2F = jnp.sum(gF * gF, axis=0, keepdims=True)
                p1C = jnp.sum(gC, axis=0, keepdims=True)
                p2C = jnp.sum(gC * gC, axis=0, keepdims=True)
                if s1F is None:
                    s1F, s2F, s1C, s2C = p1F, p2F, p1C, p2C
                else:
                    s1F, s2F, s1C, s2C = s1F + p1F, s2F + p2F, s1C + p1C, s2C + p2C
            upd = jnp.concatenate([s1F, s2F, s1C, s2C], axis=0)

            @pl.when(i == 0)
            def _():
                st_s[...] = jnp.zeros_like(st_s)

            st_s[...] += upd

        @pl.when(k == 1)
        def _phase_apply():
            stf = jnp.dot(st_s[...], K_ref[...], preferred_element_type=jnp.float32)
            m1F = stf[0:1, :] / S
            vF = stf[1:2, :] / S - m1F * m1F
            aF = ghF_ref[...] * lax.rsqrt(vF + _EPS)
            cF = bhF_ref[...] - m1F * aF
            m1C = stf[2:3, :] / S
            vC = stf[3:4, :] / S - m1C * m1C
            aC = ghC_ref[...] * lax.rsqrt(vC + _EPS)
            cC = bhC_ref[...] - m1C * aC
            tot = None
            for gF, gC in gate_halves():
                p = jax.nn.sigmoid(gF * aF + cF) * jnp.maximum(gC * aC + cC, 0.0)
                tot = p if tot is None else tot + p
            sm = jnp.dot(tot, T_ref[...], preferred_element_type=jnp.float32)
            sm_s[pl.ds(i * _NPB, _NPB), :] = sm
            t1 = jnp.sum(sm, axis=0, keepdims=True)
            t2 = jnp.sum(sm * sm, axis=0, keepdims=True)

            @pl.when(i == 0)
            def _():
                st2_s[...] = jnp.zeros_like(st2_s)

            st2_s[...] += jnp.concatenate([t1, t2], axis=0)

        @pl.when(k == 2)
        def _phase_node():
            st_v = st2_s[...]
            m1 = st_v[0:1, :] / Sn
            v = st_v[1:2, :] / Sn - m1 * m1
            aa = go_ref[...] * lax.rsqrt(v + _EPS)
            cc = bo_ref[...] - m1 * aa
            sm = sm_s[pl.ds(i * _NPB, _NPB), :]
            na_ = jnp.maximum(atom_ref[...] + aa * sm + cc, 0.0)
            if final:
                val = jnp.sum(na_ * we_ref[...])

                @pl.when(i == 0)
                def _():
                    out_ref[...] = Sn * b0_ref[...]

                out_ref[...] += val.reshape(1, 1)
            else:
                natom_ref[...] = na_

    in_specs = [
        pl.BlockSpec((4, _NPB, 128), edge_map),
        pl.BlockSpec((2, _NPB, 128), edge_map),
        pl.BlockSpec((_NPB, HA), node_map),
        pl.BlockSpec((HA, 128), c0),
        pl.BlockSpec((HA, 128), c0),
        pl.BlockSpec((1, 128), c0),
        pl.BlockSpec((1, 128), c0),
        pl.BlockSpec((128, 128), c0),
        pl.BlockSpec((128, 128), c0),
        pl.BlockSpec((128, 128), c0),
        pl.BlockSpec((128, 128), c0),
        pl.BlockSpec((128, 128), c0),
        pl.BlockSpec((128, 128), c0),
        pl.BlockSpec((1, 128), c0),
        pl.BlockSpec((1, 128), c0),
        pl.BlockSpec((1, 128), c0),
        pl.BlockSpec((1, 128), c0),
        pl.BlockSpec((1, HA), c0),
        pl.BlockSpec((1, HA), c0),
        pl.BlockSpec((128, 128), c0),
        pl.BlockSpec((128, HA), c0),
    ]
    args = [anbr3, ef3, atom, wsF, wsC, bfF, bfC,
            wnFk, wnCk, weF0, weF1, weC0, weC1,
            ghFt, bhFt, ghCt, bhCt, go2, bo2, Kfold, Tfold]
    if final:
        werow, b0 = extras
        in_specs += [pl.BlockSpec((1, HA), c0), pl.BlockSpec((1, 1), c0)]
        args += [werow, b0]
        out_specs = pl.BlockSpec((1, 1), c0)
        out_shape = jax.ShapeDtypeStruct((1, 1), jnp.float32)
    else:
        out_specs = pl.BlockSpec((_NPB, HA), out_map)
        out_shape = jax.ShapeDtypeStruct((BN, HA), jnp.float32)

    return pl.pallas_call(
        body,
        grid=(3, grid),
        in_specs=in_specs,
        out_specs=out_specs,
        out_shape=out_shape,
        scratch_shapes=[
            pltpu.VMEM((4, 128), jnp.float32),
            pltpu.VMEM((BN, HA), jnp.float32),
            pltpu.VMEM((2, HA), jnp.float32),
        ],
        compiler_params=pltpu.CompilerParams(
            dimension_semantics=("arbitrary", "arbitrary")
        ),
    )(*args)


def kernel(node_attr, edge_attr, edge_idx, Wn, b_in, Wf, bf, gh, bh, go, bo, We, b_out):
    B, N, M = edge_idx.shape
    HA = Wn.shape[0]
    HB = edge_attr.shape[-1]
    BN = B * N
    E = BN * M
    MG = M // 4
    EG = M // 8

    f32 = jnp.float32
    eye4 = jnp.eye(4, dtype=f32)
    eye8 = jnp.eye(8, dtype=f32)

    na = node_attr.reshape(BN, 1)

    idx_off = edge_idx + (jnp.arange(B, dtype=edge_idx.dtype) * N)[:, None, None]
    idx_r = idx_off.reshape(BN, MG, 4).transpose(1, 0, 2).reshape(E)

    ef3 = (
        edge_attr.reshape(BN, EG, 8, HB).transpose(1, 0, 2, 3).reshape(EG, BN, 128)
    )

    r128 = np.arange(128)
    Kfold = jnp.asarray((r128[:, None] % HA == r128[None, :] % HA), dtype=f32)
    Tfold = jnp.asarray((r128[:, None] % HA == np.arange(HA)[None, :]), dtype=f32)

    def tile4(x):
        return jnp.concatenate([x] * 4, axis=1)

    n_layers = Wf.shape[0]
    layers = []
    for i in range(n_layers):
        Wfi = Wf[i]
        k8F = jnp.kron(eye8, Wfi[:HA, 2 * HA :].T)
        k8C = jnp.kron(eye8, Wfi[HA:, 2 * HA :].T)
        layers.append(dict(
            wsF=jnp.concatenate([Wfi[:HA, :HA].T] * 4, axis=1),
            wsC=jnp.concatenate([Wfi[HA:, :HA].T] * 4, axis=1),
            wnFk=jnp.kron(eye4, Wfi[:HA, HA : 2 * HA].T),
            wnCk=jnp.kron(eye4, Wfi[HA:, HA : 2 * HA].T),
            weF0=k8F[:, :128], weF1=k8F[:, 128:],
            weC0=k8C[:, :128], weC1=k8C[:, 128:],
            bfF=tile4(bf[i][:HA].reshape(1, HA)),
            bfC=tile4(bf[i][HA:].reshape(1, HA)),
            ghFt=tile4(gh[i][:HA].reshape(1, HA)),
            ghCt=tile4(gh[i][HA:].reshape(1, HA)),
            bhFt=tile4(bh[i][:HA].reshape(1, HA)),
            bhCt=tile4(bh[i][HA:].reshape(1, HA)),
            go2=go[i].reshape(1, HA),
            bo2=bo[i].reshape(1, HA),
        ))

    atom = _tc_init(na, Wn.reshape(1, HA), b_in.reshape(1, HA))

    out = None
    for i in range(n_layers):
        Li = layers[i]
        anbr3 = _sc_gather(atom, idx_r).reshape(MG, BN, 128)
        last = i == n_layers - 1
        extras = (We.reshape(1, HA), b_out.reshape(1, 1)) if last else None
        res = _tc_layer(
            anbr3, ef3, atom,
            Li["wsF"], Li["wsC"], Li["bfF"], Li["bfC"],
            Li["wnFk"], Li["wnCk"],
            Li["weF0"], Li["weF1"], Li["weC0"], Li["weC1"],
            Li["ghFt"], Li["bhFt"], Li["ghCt"], Li["bhCt"],
            Li["go2"], Li["bo2"], Kfold, Tfold, float(E),
            last, extras,
        )
        if last:
            out = res
        else:
            atom = res

    return out.reshape(())


# speedup vs baseline: 33.6868x; 1.0340x over previous
import functools

import jax
import jax.numpy as jnp
import numpy as np
from jax import lax
from jax.experimental import pallas as pl
from jax.experimental.pallas import tpu as pltpu
from jax.experimental.pallas import tpu_sc as plsc

_EPS = 1e-5
_NPB = 800
_NPB2 = 2000


def _sc_gather(table, idx):
    V, D = table.shape
    E = idx.shape[0]
    info = plsc.get_sparse_core_info()
    NC, NS = info.num_cores, info.num_subcores
    NW = NC * NS
    assert E % NW == 0
    e_per_w = E // NW
    CH = 2000
    assert e_per_w % CH == 0
    n_ch = e_per_w // CH
    mesh = plsc.VectorSubcoreMesh(core_axis_name="c", subcore_axis_name="s")

    @functools.partial(
        pl.kernel,
        mesh=mesh,
        out_type=jax.ShapeDtypeStruct((E, D), jnp.float32),
        compiler_params=pltpu.CompilerParams(use_tc_tiling_on_sc=False),
        scratch_types=[
            pltpu.VMEM((CH,), jnp.int32),
            pltpu.VMEM((CH, D), jnp.float32),
            pltpu.SemaphoreType.DMA,
        ],
    )
    def k(table_hbm, idx_hbm, out_hbm, idx_v, rows_v, sem):
        wid = lax.axis_index("s") * NC + lax.axis_index("c")
        base = wid * e_per_w

        def body(i, carry):
            off = base + i * CH
            pltpu.sync_copy(idx_hbm.at[pl.ds(off, CH)], idx_v)
            pltpu.async_copy(table_hbm.at[idx_v], rows_v, sem).wait()
            pltpu.sync_copy(rows_v, out_hbm.at[pl.ds(off, CH)])
            return carry

        lax.fori_loop(0, n_ch, body, 0)

    return k(table, idx)


def _tc_init(na, wnrow, binrow):
    BN = na.shape[0]
    HA = wnrow.shape[1]
    grid = BN // _NPB2

    def body(na_ref, w_ref, b_ref, atom_ref):
        atom_ref[...] = na_ref[...] * w_ref[...] + b_ref[...]

    return pl.pallas_call(
        body,
        grid=(grid,),
        in_specs=[
            pl.BlockSpec((_NPB2, 1), lambda i: (i, 0)),
            pl.BlockSpec((1, HA), lambda i: (0, 0)),
            pl.BlockSpec((1, HA), lambda i: (0, 0)),
        ],
        out_specs=pl.BlockSpec((_NPB2, HA), lambda i: (i, 0)),
        out_shape=jax.ShapeDtypeStruct((BN, HA), jnp.float32),
    )(na, wnrow, binrow)


def _tc_layer(anbr3, ef3, atom, wsF, wsC, bfF, bfC,
              wnFk, wnCk, weF0, weF1, weC0, weC1,
              ghFt, bhFt, ghCt, bhCt, go2, bo2, Kfold, Tfold, S,
              final, extras):
    BN, HA = atom.shape
    grid = BN // _NPB
    Sn = float(BN)

    c0 = lambda k, i: (0, 0)
    edge_map = lambda k, i: (0, jnp.where(k < 2, i, 0), 0)
    node_map = lambda k, i: (i, 0)
    out_map = lambda k, i: (jnp.where(k == 2, i, 0), 0)

    def body(a_ref, e_ref, atom_ref, wsF_ref, wsC_ref, bfF_ref, bfC_ref,
             wnF_ref, wnC_ref, weF0_ref, weF1_ref, weC0_ref, weC1_ref,
             ghF_ref, bhF_ref, ghC_ref, bhC_ref,
             go_ref, bo_ref, K_ref, T_ref, *rest):
        if final:
            we_ref, b0_ref, out_ref, st_s, sm_s, st2_s = rest
        else:
            natom_ref, st_s, sm_s, st2_s = rest
        k = pl.program_id(0)
        i = pl.program_id(1)

        def gate_halves():
            at = atom_ref[...]
            sF = (
                jnp.dot(at, wsF_ref[...], preferred_element_type=jnp.float32)
                + bfF_ref[...]
            )
            sC = (
                jnp.dot(at, wsC_ref[...], preferred_element_type=jnp.float32)
                + bfC_ref[...]
            )
            eh = [e_ref[0], e_ref[1]]
            weF = [weF0_ref[...], weF1_ref[...]]
            weC = [weC0_ref[...], weC1_ref[...]]
            outs = []
            for kk in range(4):
                a2 = a_ref[kk]
                e2 = eh[kk // 2]
                gF = (
                    jnp.dot(a2, wnF_ref[...], preferred_element_type=jnp.float32)
                    + jnp.dot(e2, weF[kk % 2], preferred_element_type=jnp.float32)
                    + sF
                )
                gC = (
                    jnp.dot(a2, wnC_ref[...], preferred_element_type=jnp.float32)
                    + jnp.dot(e2, weC[kk % 2], preferred_element_type=jnp.float32)
                    + sC
                )
                outs.append((gF, gC))
            return outs

        @pl.when(k == 0)
        def _phase_stats():
            s1F = s2F = s1C = s2C = None
            for gF, gC in gate_halves():
                p1F = jnp.sum(gF, axis=0, keepdims=True)
                p2F = jnp.sum(gF * gF, axis=0, keepdims=True)
                p1C = jnp.sum(gC, axis=0, keepdims=True)
                p2C = jnp.sum(gC * gC, axis=0, keepdims=True)
                if s1F is None:
                    s1F, s2F, s1C, s2C = p1F, p2F, p1C, p2C
                else:
                    s1F, s2F, s1C, s2C = s1F + p1F, s2F + p2F, s1C + p1C, s2C + p2C
            upd = jnp.concatenate([s1F, s2F, s1C, s2C], axis=0)

            @pl.when(i == 0)
            def _():
                st_s[...] = jnp.zeros_like(st_s)

            st_s[...] += upd

        @pl.when(k == 1)
        def _phase_apply():
            stf = jnp.dot(st_s[...], K_ref[...], preferred_element_type=jnp.float32)
            m1F = stf[0:1, :] / S
            vF = stf[1:2, :] / S - m1F * m1F
            aF = ghF_ref[...] * lax.rsqrt(vF + _EPS)
            cF = bhF_ref[...] - m1F * aF
            m1C = stf[2:3, :] / S
            vC = stf[3:4, :] / S - m1C * m1C
            aC = ghC_ref[...] * lax.rsqrt(vC + _EPS)
            cC = bhC_ref[...] - m1C * aC
            tot = None
            for gF, gC in gate_halves():
                p = jax.nn.sigmoid(gF * aF + cF) * jnp.maximum(gC * aC + cC, 0.0)
                tot = p if tot is None else tot + p
            sm = jnp.dot(tot, T_ref[...], preferred_element_type=jnp.float32)
            sm_s[pl.ds(i * _NPB, _NPB), :] = sm
            t1 = jnp.sum(sm, axis=0, keepdims=True)
            t2 = jnp.sum(sm * sm, axis=0, keepdims=True)

            @pl.when(i == 0)
            def _():
                st2_s[...] = jnp.zeros_like(st2_s)

            st2_s[...] += jnp.concatenate([t1, t2], axis=0)

        @pl.when(k == 2)
        def _phase_node():
            st_v = st2_s[...]
            m1 = st_v[0:1, :] / Sn
            v = st_v[1:2, :] / Sn - m1 * m1
            aa = go_ref[...] * lax.rsqrt(v + _EPS)
            cc = bo_ref[...] - m1 * aa
            sm = sm_s[pl.ds(i * _NPB, _NPB), :]
            na_ = jnp.maximum(atom_ref[...] + aa * sm + cc, 0.0)
            if final:
                val = jnp.sum(na_ * we_ref[...])

                @pl.when(i == 0)
                def _():
                    out_ref[...] = Sn * b0_ref[...]

                out_ref[...] += val.reshape(1, 1)
            else:
                natom_ref[...] = na_

    in_specs = [
        pl.BlockSpec((4, _NPB, 128), edge_map),
        pl.BlockSpec((2, _NPB, 128), edge_map),
        pl.BlockSpec((_NPB, HA), node_map),
        pl.BlockSpec((HA, 128), c0),
        pl.BlockSpec((HA, 128), c0),
        pl.BlockSpec((1, 128), c0),
        pl.BlockSpec((1, 128), c0),
        pl.BlockSpec((128, 128), c0),
        pl.BlockSpec((128, 128), c0),
        pl.BlockSpec((128, 128), c0),
        pl.BlockSpec((128, 128), c0),
        pl.BlockSpec((128, 128), c0),
        pl.BlockSpec((128, 128), c0),
        pl.BlockSpec((1, 128), c0),
        pl.BlockSpec((1, 128), c0),
        pl.BlockSpec((1, 128), c0),
        pl.BlockSpec((1, 128), c0),
        pl.BlockSpec((1, HA), c0),
        pl.BlockSpec((1, HA), c0),
        pl.BlockSpec((128, 128), c0),
        pl.BlockSpec((128, HA), c0),
    ]
    args = [anbr3, ef3, atom, wsF, wsC, bfF, bfC,
            wnFk, wnCk, weF0, weF1, weC0, weC1,
            ghFt, bhFt, ghCt, bhCt, go2, bo2, Kfold, Tfold]
    if final:
        werow, b0 = extras
        in_specs += [pl.BlockSpec((1, HA), c0), pl.BlockSpec((1, 1), c0)]
        args += [werow, b0]
        out_specs = pl.BlockSpec((1, 1), c0)
        out_shape = jax.ShapeDtypeStruct((1, 1), jnp.float32)
    else:
        out_specs = pl.BlockSpec((_NPB, HA), out_map)
        out_shape = jax.ShapeDtypeStruct((BN, HA), jnp.float32)

    return pl.pallas_call(
        body,
        grid=(3, grid),
        in_specs=in_specs,
        out_specs=out_specs,
        out_shape=out_shape,
        scratch_shapes=[
            pltpu.VMEM((4, 128), jnp.float32),
            pltpu.VMEM((BN, HA), jnp.float32),
            pltpu.VMEM((2, HA), jnp.float32),
        ],
        compiler_params=pltpu.CompilerParams(
            dimension_semantics=("arbitrary", "arbitrary")
        ),
    )(*args)


def kernel(node_attr, edge_attr, edge_idx, Wn, b_in, Wf, bf, gh, bh, go, bo, We, b_out):
    B, N, M = edge_idx.shape
    HA = Wn.shape[0]
    HB = edge_attr.shape[-1]
    BN = B * N
    E = BN * M
    MG = M // 4
    EG = M // 8

    f32 = jnp.float32
    eye4 = jnp.eye(4, dtype=f32)
    eye8 = jnp.eye(8, dtype=f32)

    na = node_attr.reshape(BN, 1)

    idx_off = edge_idx + (jnp.arange(B, dtype=edge_idx.dtype) * N)[:, None, None]
    idx_r = idx_off.reshape(BN, MG, 4).transpose(1, 0, 2).reshape(E)

    ef3 = (
        edge_attr.reshape(BN, EG, 8, HB).transpose(1, 0, 2, 3).reshape(EG, BN, 128)
    ).astype(jnp.bfloat16)

    r128 = np.arange(128)
    Kfold = jnp.asarray((r128[:, None] % HA == r128[None, :] % HA), dtype=f32)
    Tfold = jnp.asarray((r128[:, None] % HA == np.arange(HA)[None, :]), dtype=f32)

    def tile4(x):
        return jnp.concatenate([x] * 4, axis=1)

    n_layers = Wf.shape[0]
    layers = []
    for i in range(n_layers):
        Wfi = Wf[i]
        k8F = jnp.kron(eye8, Wfi[:HA, 2 * HA :].T)
        k8C = jnp.kron(eye8, Wfi[HA:, 2 * HA :].T)
        layers.append(dict(
            wsF=jnp.concatenate([Wfi[:HA, :HA].T] * 4, axis=1),
            wsC=jnp.concatenate([Wfi[HA:, :HA].T] * 4, axis=1),
            wnFk=jnp.kron(eye4, Wfi[:HA, HA : 2 * HA].T),
            wnCk=jnp.kron(eye4, Wfi[HA:, HA : 2 * HA].T),
            weF0=k8F[:, :128].astype(jnp.bfloat16),
            weF1=k8F[:, 128:].astype(jnp.bfloat16),
            weC0=k8C[:, :128].astype(jnp.bfloat16),
            weC1=k8C[:, 128:].astype(jnp.bfloat16),
            bfF=tile4(bf[i][:HA].reshape(1, HA)),
            bfC=tile4(bf[i][HA:].reshape(1, HA)),
            ghFt=tile4(gh[i][:HA].reshape(1, HA)),
            ghCt=tile4(gh[i][HA:].reshape(1, HA)),
            bhFt=tile4(bh[i][:HA].reshape(1, HA)),
            bhCt=tile4(bh[i][HA:].reshape(1, HA)),
            go2=go[i].reshape(1, HA),
            bo2=bo[i].reshape(1, HA),
        ))

    atom = _tc_init(na, Wn.reshape(1, HA), b_in.reshape(1, HA))

    out = None
    for i in range(n_layers):
        Li = layers[i]
        anbr3 = _sc_gather(atom, idx_r).reshape(MG, BN, 128)
        last = i == n_layers - 1
        extras = (We.reshape(1, HA), b_out.reshape(1, 1)) if last else None
        res = _tc_layer(
            anbr3, ef3, atom,
            Li["wsF"], Li["wsC"], Li["bfF"], Li["bfC"],
            Li["wnFk"], Li["wnCk"],
            Li["weF0"], Li["weF1"], Li["weC0"], Li["weC1"],
            Li["ghFt"], Li["bhFt"], Li["ghCt"], Li["bhCt"],
            Li["go2"], Li["bo2"], Kfold, Tfold, float(E),
            last, extras,
        )
        if last:
            out = res
        else:
            atom = res

    return out.reshape(())
